# baseline (speedup 1.0000x reference)
import functools

import jax
import jax.numpy as jnp
from jax.experimental import pallas as pl
from jax.experimental.pallas import tpu as pltpu
from jax.experimental.pallas import tpu_sc as plsc

_HIGH = jax.lax.Precision.HIGHEST


def _proj_body(nfeat_ref, w1s_ref, w1d_ref, ps_ref, pd_ref):
    x = nfeat_ref[...]
    ps_ref[...] = jax.lax.dot(x, w1s_ref[...], precision=_HIGH,
                              preferred_element_type=jnp.float32)
    pd_ref[...] = jax.lax.dot(x, w1d_ref[...], precision=_HIGH,
                              preferred_element_type=jnp.float32)


def _project_nodes(nfeat, w1s, w1d):
    n = nfeat.shape[0]
    out = jax.ShapeDtypeStruct((n, w1s.shape[1]), jnp.float32)
    return pl.pallas_call(
        _proj_body,
        out_shape=(out, out),
    )(nfeat, w1s, w1d)


def _gather_pairs(ps, pd, src2d, dst2d, gather_window):
    e = src2d.shape[1]
    d = ps.shape[1]
    out = jax.ShapeDtypeStruct((e, d), ps.dtype)
    mesh = plsc.VectorSubcoreMesh(core_axis_name="core",
                                  subcore_axis_name="subcore")

    @pl.kernel(out_type=out, mesh=mesh)
    def kern(ps_hbm, pd_hbm, src_hbm, dst_hbm, g_hbm):
        def body(src_vmem, dst_vmem, g_vmem):
            pltpu.sync_copy(ps_hbm.at[src_vmem.at[0]], g_vmem)
            pltpu.sync_copy(pd_hbm.at[dst_vmem.at[0]], g_vmem, add=True)

        pltpu.emit_pipeline(
            body,
            grid=(e // gather_window,),
            in_specs=[pl.BlockSpec((1, gather_window), lambda i: (0, i)),
                      pl.BlockSpec((1, gather_window), lambda i: (0, i))],
            out_specs=[pl.BlockSpec((gather_window, d), lambda i: (i, 0))],
            core_axis_name=("core", "subcore"),
            dimension_semantics=(pltpu.PARALLEL,),
        )(src_hbm, dst_hbm, g_hbm)

    return kern(ps, pd, src2d, dst2d)


def _mlp_body(efeat_ref, g_ref, w1e_ref, b1_ref, w2_ref, b2_ref,
              lng_ref, lnb_ref, out_ref):
    x = efeat_ref[...]
    pre = jax.lax.dot(x, w1e_ref[...],
                      preferred_element_type=jnp.float32)
    pre = pre + g_ref[...] + b1_ref[...]
    h = pre * jax.lax.logistic(pre)
    y = jax.lax.dot(h, w2_ref[...],
                    preferred_element_type=jnp.float32) + b2_ref[...]
    mu = jnp.mean(y, axis=-1, keepdims=True)
    yc = y - mu
    var = jnp.mean(yc * yc, axis=-1, keepdims=True)
    out = yc * jax.lax.rsqrt(var + 1e-5) * lng_ref[...] + lnb_ref[...]
    out_ref[...] = out + x


def _mlp_body_buf(buf_ref, efeat_ref, g_ref, w1e_ref, b1_ref,
                  w2_ref, b2_ref, lng_ref, lnb_ref, out_ref):
    del buf_ref
    _mlp_body(efeat_ref, g_ref, w1e_ref, b1_ref, w2_ref, b2_ref,
              lng_ref, lnb_ref, out_ref)


def _edge_mlp_chunk(buf, efeat, g, w1e, b1, w2, b2, ln_g, ln_b,
                    block_e, off_blocks):
    e, d = efeat.shape
    chunk = g.shape[0]
    dh = w1e.shape[1]
    grid = (chunk // block_e,)
    edge_spec = pl.BlockSpec((block_e, d),
                             lambda i, o=off_blocks: (i + o, 0))
    g_spec = pl.BlockSpec((block_e, dh), lambda i: (i, 0))
    w_spec = lambda r, c: pl.BlockSpec((r, c), lambda i: (0, 0))
    in_specs = [edge_spec, g_spec,
                w_spec(d, dh), w_spec(1, dh),
                w_spec(dh, w2.shape[1]), w_spec(1, w2.shape[1]),
                w_spec(1, w2.shape[1]), w_spec(1, w2.shape[1])]
    operands = [efeat, g, w1e, b1.reshape(1, -1), w2,
                b2.reshape(1, -1), ln_g.reshape(1, -1), ln_b.reshape(1, -1)]
    body = _mlp_body
    aliases = {}
    if buf is not None:
        in_specs = [pl.BlockSpec(memory_space=pltpu.MemorySpace.HBM)
                    ] + in_specs
        operands = [buf] + operands
        body = _mlp_body_buf
        aliases = {0: 0}
    return pl.pallas_call(
        body,
        grid=grid,
        in_specs=in_specs,
        out_specs=edge_spec,
        out_shape=jax.ShapeDtypeStruct((e, d), jnp.float32),
        input_output_aliases=aliases,
        compiler_params=pltpu.CompilerParams(
            dimension_semantics=("parallel",)),
    )(*operands)


@functools.partial(jax.jit, static_argnames=())
def kernel(efeat, nfeat, edge_index, W1, b1, W2, b2, ln_g, ln_b):
    e, d_edge = efeat.shape
    d_node = nfeat.shape[1]
    w1e = W1[:d_edge]
    w1s = W1[d_edge:d_edge + d_node]
    w1d = W1[d_edge + d_node:]

    ps, pd = _project_nodes(nfeat, w1s, w1d)

    idx = edge_index.astype(jnp.int32)

    block_e = 1600
    n_blocks = e // block_e
    chunk_blocks = [26, 26, 24, 24, 26, 26, 24, 24]
    assert sum(chunk_blocks) == n_blocks
    buf = None
    off = 0
    for cb in chunk_blocks:
        lo, hi = off * block_e, (off + cb) * block_e
        src2d = jax.lax.slice(idx, (0, lo), (1, hi))
        dst2d = jax.lax.slice(idx, (1, lo), (2, hi))
        g = _gather_pairs(ps, pd, src2d, dst2d, gather_window=128)
        buf = _edge_mlp_chunk(buf, efeat, g, w1e, b1, W2, b2,
                              ln_g, ln_b,
                              block_e=block_e, off_blocks=off)
        off += cb
    return (buf, nfeat)


# speedup vs baseline: 1.0149x; 1.0149x over previous
import functools

import jax
import jax.numpy as jnp
from jax.experimental import pallas as pl
from jax.experimental.pallas import tpu as pltpu
from jax.experimental.pallas import tpu_sc as plsc

_HIGH = jax.lax.Precision.HIGHEST


def _proj_body(nfeat_ref, w1s_ref, w1d_ref, ps_ref, pd_ref):
    x = nfeat_ref[...]
    ps_ref[...] = jax.lax.dot(x, w1s_ref[...], precision=_HIGH,
                              preferred_element_type=jnp.float32)
    pd_ref[...] = jax.lax.dot(x, w1d_ref[...], precision=_HIGH,
                              preferred_element_type=jnp.float32)


def _project_nodes(nfeat, w1s, w1d):
    n = nfeat.shape[0]
    out = jax.ShapeDtypeStruct((n, w1s.shape[1]), jnp.float32)
    return pl.pallas_call(
        _proj_body,
        out_shape=(out, out),
    )(nfeat, w1s, w1d)


def _gather_pairs(ps, pd, src2d, dst2d, gather_window):
    e = src2d.shape[1]
    d = ps.shape[1]
    out = jax.ShapeDtypeStruct((e, d), ps.dtype)
    mesh = plsc.VectorSubcoreMesh(core_axis_name="core",
                                  subcore_axis_name="subcore")

    @pl.kernel(out_type=out, mesh=mesh)
    def kern(ps_hbm, pd_hbm, src_hbm, dst_hbm, g_hbm):
        def body(src_vmem, dst_vmem, g_vmem):
            pltpu.sync_copy(ps_hbm.at[src_vmem.at[0]], g_vmem)
            pltpu.sync_copy(pd_hbm.at[dst_vmem.at[0]], g_vmem, add=True)

        pltpu.emit_pipeline(
            body,
            grid=(e // gather_window,),
            in_specs=[pl.BlockSpec((1, gather_window), lambda i: (0, i)),
                      pl.BlockSpec((1, gather_window), lambda i: (0, i))],
            out_specs=[pl.BlockSpec((gather_window, d), lambda i: (i, 0))],
            core_axis_name=("core", "subcore"),
            dimension_semantics=(pltpu.PARALLEL,),
        )(src_hbm, dst_hbm, g_hbm)

    return kern(ps, pd, src2d, dst2d)


def _mlp_body(efeat_ref, g_ref, w1e_ref, b1_ref, w2_ref, b2_ref,
              lng_ref, lnb_ref, out_ref):
    x = efeat_ref[...]
    pre = jax.lax.dot(x, w1e_ref[...],
                      preferred_element_type=jnp.float32)
    pre = pre + g_ref[...] + b1_ref[...]
    h = pre * jax.lax.logistic(pre)
    y = jax.lax.dot(h, w2_ref[...],
                    preferred_element_type=jnp.float32) + b2_ref[...]
    mu = jnp.mean(y, axis=-1, keepdims=True)
    yc = y - mu
    var = jnp.mean(yc * yc, axis=-1, keepdims=True)
    out = yc * jax.lax.rsqrt(var + 1e-5) * lng_ref[...] + lnb_ref[...]
    out_ref[...] = out + x


def _mlp_body_buf(buf_ref, efeat_ref, g_ref, w1e_ref, b1_ref,
                  w2_ref, b2_ref, lng_ref, lnb_ref, out_ref):
    del buf_ref
    _mlp_body(efeat_ref, g_ref, w1e_ref, b1_ref, w2_ref, b2_ref,
              lng_ref, lnb_ref, out_ref)


def _edge_mlp_chunk(buf, efeat, g, w1e, b1, w2, b2, ln_g, ln_b,
                    block_e, off_blocks):
    e, d = efeat.shape
    chunk = g.shape[0]
    dh = w1e.shape[1]
    grid = (chunk // block_e,)
    edge_spec = pl.BlockSpec((block_e, d),
                             lambda i, o=off_blocks: (i + o, 0))
    g_spec = pl.BlockSpec((block_e, dh), lambda i: (i, 0))
    w_spec = lambda r, c: pl.BlockSpec((r, c), lambda i: (0, 0))
    in_specs = [edge_spec, g_spec,
                w_spec(d, dh), w_spec(1, dh),
                w_spec(dh, w2.shape[1]), w_spec(1, w2.shape[1]),
                w_spec(1, w2.shape[1]), w_spec(1, w2.shape[1])]
    operands = [efeat, g, w1e, b1.reshape(1, -1), w2,
                b2.reshape(1, -1), ln_g.reshape(1, -1), ln_b.reshape(1, -1)]
    body = _mlp_body
    aliases = {}
    if buf is not None:
        in_specs = [pl.BlockSpec(memory_space=pltpu.MemorySpace.HBM)
                    ] + in_specs
        operands = [buf] + operands
        body = _mlp_body_buf
        aliases = {0: 0}
    return pl.pallas_call(
        body,
        grid=grid,
        in_specs=in_specs,
        out_specs=edge_spec,
        out_shape=jax.ShapeDtypeStruct((e, d), jnp.float32),
        input_output_aliases=aliases,
        compiler_params=pltpu.CompilerParams(
            dimension_semantics=("parallel",)),
    )(*operands)


@functools.partial(jax.jit, static_argnames=())
def kernel(efeat, nfeat, edge_index, W1, b1, W2, b2, ln_g, ln_b):
    e, d_edge = efeat.shape
    d_node = nfeat.shape[1]
    w1e = W1[:d_edge]
    w1s = W1[d_edge:d_edge + d_node]
    w1d = W1[d_edge + d_node:]

    ps, pd = _project_nodes(nfeat, w1s, w1d)

    idx = edge_index.astype(jnp.int32)

    block_e = 1600
    n_blocks = e // block_e
    chunk_blocks = [34, 34, 34, 34, 32, 32]
    assert sum(chunk_blocks) == n_blocks
    buf = None
    off = 0
    for cb in chunk_blocks:
        lo, hi = off * block_e, (off + cb) * block_e
        src2d = jax.lax.slice(idx, (0, lo), (1, hi))
        dst2d = jax.lax.slice(idx, (1, lo), (2, hi))
        g = _gather_pairs(ps, pd, src2d, dst2d, gather_window=128)
        buf = _edge_mlp_chunk(buf, efeat, g, w1e, b1, W2, b2,
                              ln_g, ln_b,
                              block_e=block_e, off_blocks=off)
        off += cb
    return (buf, nfeat)


# speedup vs baseline: 1.0364x; 1.0212x over previous
import functools

import jax
import jax.numpy as jnp
from jax.experimental import pallas as pl
from jax.experimental.pallas import tpu as pltpu
from jax.experimental.pallas import tpu_sc as plsc

_HIGH = jax.lax.Precision.HIGHEST


def _proj_body(nfeat_ref, w1s_ref, w1d_ref, ps_ref, pd_ref):
    x = nfeat_ref[...]
    ps_ref[...] = jax.lax.dot(x, w1s_ref[...], precision=_HIGH,
                              preferred_element_type=jnp.float32)
    pd_ref[...] = jax.lax.dot(x, w1d_ref[...], precision=_HIGH,
                              preferred_element_type=jnp.float32)


def _project_nodes(nfeat, w1_full, d_edge):
    n, d_node = nfeat.shape
    dh = w1_full.shape[1]
    out = jax.ShapeDtypeStruct((n, dh), jnp.float32)
    assert d_edge % d_node == 0
    sb = d_edge // d_node
    return pl.pallas_call(
        _proj_body,
        grid=(1,),
        in_specs=[pl.BlockSpec((n, d_node), lambda i: (0, 0)),
                  pl.BlockSpec((d_node, dh), lambda i, s=sb: (s, 0)),
                  pl.BlockSpec((d_node, dh), lambda i, s=sb: (s + 1, 0))],
        out_specs=(pl.BlockSpec((n, dh), lambda i: (0, 0)),
                   pl.BlockSpec((n, dh), lambda i: (0, 0))),
        out_shape=(out, out),
    )(nfeat, w1_full, w1_full)


def _gather_pairs(ps, pd, idx, chunk, off_edges, gather_window):
    d = ps.shape[1]
    offw = off_edges // gather_window
    out = jax.ShapeDtypeStruct((chunk, d), ps.dtype)
    mesh = plsc.VectorSubcoreMesh(core_axis_name="core",
                                  subcore_axis_name="subcore")

    @pl.kernel(out_type=out, mesh=mesh)
    def kern(ps_hbm, pd_hbm, idx_hbm, g_hbm):
        def body(src_vmem, dst_vmem, g_vmem):
            pltpu.sync_copy(ps_hbm.at[src_vmem.at[0]], g_vmem)
            pltpu.sync_copy(pd_hbm.at[dst_vmem.at[0]], g_vmem, add=True)

        pltpu.emit_pipeline(
            body,
            grid=(chunk // gather_window,),
            in_specs=[pl.BlockSpec((1, gather_window),
                                   lambda i, o=offw: (0, i + o)),
                      pl.BlockSpec((1, gather_window),
                                   lambda i, o=offw: (1, i + o))],
            out_specs=[pl.BlockSpec((gather_window, d), lambda i: (i, 0))],
            core_axis_name=("core", "subcore"),
            dimension_semantics=(pltpu.PARALLEL,),
        )(idx_hbm, idx_hbm, g_hbm)

    return kern(ps, pd, idx)


def _mlp_body(efeat_ref, g_ref, w1e_ref, b1_ref, w2_ref, b2_ref,
              lng_ref, lnb_ref, out_ref):
    x = efeat_ref[...]
    pre = jax.lax.dot(x, w1e_ref[...],
                      preferred_element_type=jnp.float32)
    pre = pre + g_ref[...] + b1_ref[...]
    h = pre * jax.lax.logistic(pre)
    y = jax.lax.dot(h, w2_ref[...],
                    preferred_element_type=jnp.float32) + b2_ref[...]
    mu = jnp.mean(y, axis=-1, keepdims=True)
    yc = y - mu
    var = jnp.mean(yc * yc, axis=-1, keepdims=True)
    out = yc * jax.lax.rsqrt(var + 1e-5) * lng_ref[...] + lnb_ref[...]
    out_ref[...] = out + x


def _mlp_body_buf(buf_ref, efeat_ref, g_ref, w1e_ref, b1_ref,
                  w2_ref, b2_ref, lng_ref, lnb_ref, out_ref):
    del buf_ref
    _mlp_body(efeat_ref, g_ref, w1e_ref, b1_ref, w2_ref, b2_ref,
              lng_ref, lnb_ref, out_ref)


def _edge_mlp_chunk(buf, efeat, g, w1e, b1, w2, b2, ln_g, ln_b,
                    block_e, off_blocks):
    e, d = efeat.shape
    chunk = g.shape[0]
    dh = w1e.shape[1]
    grid = (chunk // block_e,)
    edge_spec = pl.BlockSpec((block_e, d),
                             lambda i, o=off_blocks: (i + o, 0))
    g_spec = pl.BlockSpec((block_e, dh), lambda i: (i, 0))
    w_spec = lambda r, c: pl.BlockSpec((r, c), lambda i: (0, 0))
    in_specs = [edge_spec, g_spec,
                pl.BlockSpec((d, dh), lambda i: (0, 0)), w_spec(1, dh),
                w_spec(dh, w2.shape[1]), w_spec(1, w2.shape[1]),
                w_spec(1, w2.shape[1]), w_spec(1, w2.shape[1])]
    operands = [efeat, g, w1e, b1.reshape(1, -1), w2,
                b2.reshape(1, -1), ln_g.reshape(1, -1), ln_b.reshape(1, -1)]
    body = _mlp_body
    aliases = {}
    if buf is not None:
        in_specs = [pl.BlockSpec(memory_space=pltpu.MemorySpace.HBM)
                    ] + in_specs
        operands = [buf] + operands
        body = _mlp_body_buf
        aliases = {0: 0}
    return pl.pallas_call(
        body,
        grid=grid,
        in_specs=in_specs,
        out_specs=edge_spec,
        out_shape=jax.ShapeDtypeStruct((e, d), jnp.float32),
        input_output_aliases=aliases,
        compiler_params=pltpu.CompilerParams(
            dimension_semantics=("parallel",)),
    )(*operands)


@functools.partial(jax.jit, static_argnames=())
def kernel(efeat, nfeat, edge_index, W1, b1, W2, b2, ln_g, ln_b):
    e, d_edge = efeat.shape

    ps, pd = _project_nodes(nfeat, W1, d_edge)

    idx = edge_index.astype(jnp.int32)

    block_e = 1600
    n_blocks = e // block_e
    chunk_blocks = [40, 40, 40, 40, 40]
    assert sum(chunk_blocks) == n_blocks
    buf = None
    off = 0
    for cb in chunk_blocks:
        g = _gather_pairs(ps, pd, idx, chunk=cb * block_e,
                          off_edges=off * block_e, gather_window=128)
        buf = _edge_mlp_chunk(buf, efeat, g, W1, b1, W2, b2,
                              ln_g, ln_b,
                              block_e=block_e, off_blocks=off)
        off += cb
    return (buf, nfeat)
